# baseline (speedup 1.0000x reference)
import functools

import jax
import jax.numpy as jnp
from jax import lax
from jax.experimental import pallas as pl
from jax.experimental.pallas import tpu as pltpu


def _topk_mean_bits(a_abs, k):
    u = lax.bitcast_convert_type(a_abs, jnp.int32)
    t = jnp.int32(0)
    for bit in range(30, -1, -1):
        cand = t | jnp.int32(1 << bit)
        cnt = jnp.sum((u >= cand).astype(jnp.int32))
        t = jnp.where(cnt >= k, cand, t)
    kth = lax.bitcast_convert_type(t, jnp.float32)
    gt = u > t
    cnt_gt = jnp.sum(gt.astype(jnp.int32))
    sum_gt = jnp.sum(jnp.where(gt, a_abs, jnp.float32(0.0)))
    total = sum_gt + (jnp.float32(k) - cnt_gt.astype(jnp.float32)) * kth
    return total / jnp.float32(k)


def _body(k, nb, x_ref, w_ref, b_ref, o_ref):
    i = pl.program_id(0)
    xb = x_ref[0]
    w = w_ref[...]
    s = lax.dot_general(
        w, xb, (((1,), (0,)), ((), ())),
        preferred_element_type=jnp.float32,
    )
    s = s + b_ref[0]
    m = _topk_mean_bits(jnp.abs(s), k)

    @pl.when(i == 0)
    def _():
        o_ref[...] = jnp.zeros_like(o_ref)

    row = lax.broadcasted_iota(jnp.int32, (nb, 1), 0)
    o_ref[...] += jnp.where(row == i, m, jnp.float32(0.0))


def kernel(x, W, b):
    B, C, H, Wd = x.shape
    HW = H * Wd
    k = max(int(HW * 0.1), 1)
    xr = x.reshape(B, C, HW)
    wv = W.reshape(1, C)
    out = pl.pallas_call(
        functools.partial(_body, k, B),
        grid=(B,),
        in_specs=[
            pl.BlockSpec((1, C, HW), lambda i: (i, 0, 0)),
            pl.BlockSpec((1, C), lambda i: (0, 0)),
            pl.BlockSpec(memory_space=pltpu.SMEM),
        ],
        out_specs=pl.BlockSpec((B, 1), lambda i: (0, 0)),
        out_shape=jax.ShapeDtypeStruct((B, 1), jnp.float32),
    )(xr, wv, b)
    return out


# speedup vs baseline: 1.9195x; 1.9195x over previous
import functools

import jax
import jax.numpy as jnp
from jax import lax
from jax.experimental import pallas as pl
from jax.experimental.pallas import tpu as pltpu


def _topk_mean_rows(a_abs, k):
    u = lax.bitcast_convert_type(a_abs, jnp.int32)
    t = jnp.zeros((a_abs.shape[0], 1), jnp.int32)
    for bit in range(30, -1, -1):
        cand = t | jnp.int32(1 << bit)
        cnt = jnp.sum((u >= cand).astype(jnp.int32), axis=1, keepdims=True)
        t = jnp.where(cnt >= k, cand, t)
    kth = lax.bitcast_convert_type(t, jnp.float32)
    gt = u > t
    cnt_gt = jnp.sum(gt.astype(jnp.int32), axis=1, keepdims=True)
    sum_gt = jnp.sum(jnp.where(gt, a_abs, jnp.float32(0.0)), axis=1,
                     keepdims=True)
    total = sum_gt + (jnp.float32(k) - cnt_gt.astype(jnp.float32)) * kth
    return total / jnp.float32(k)


def _body(k, bblk, x_ref, w_ref, b_ref, o_ref):
    xb = x_ref[...]
    w = w_ref[...]
    wb = jnp.broadcast_to(w[None, :, :], (bblk, 1, w.shape[1]))
    s = lax.dot_general(
        wb, xb, (((2,), (1,)), ((0,), (0,))),
        preferred_element_type=jnp.float32,
    )
    s = s[:, 0, :] + b_ref[0]
    o_ref[...] = _topk_mean_rows(jnp.abs(s), k)


def kernel(x, W, b):
    B, C, H, Wd = x.shape
    HW = H * Wd
    k = max(int(HW * 0.1), 1)
    bblk = 8
    xr = x.reshape(B, C, HW)
    wv = W.reshape(1, C)
    out = pl.pallas_call(
        functools.partial(_body, k, bblk),
        grid=(B // bblk,),
        in_specs=[
            pl.BlockSpec((bblk, C, HW), lambda i: (i, 0, 0)),
            pl.BlockSpec((1, C), lambda i: (0, 0)),
            pl.BlockSpec(memory_space=pltpu.SMEM),
        ],
        out_specs=pl.BlockSpec((bblk, 1), lambda i: (i, 0)),
        out_shape=jax.ShapeDtypeStruct((B, 1), jnp.float32),
    )(xr, wv, b)
    return out
